# baseline (speedup 1.0000x reference)
import functools

import jax
import jax.numpy as jnp
from jax.experimental import pallas as pl
from jax.experimental.pallas import tpu as pltpu

N = 10000

BM = 400
BH = BM // 2
N_I = N // BM


def _bf(x):
    return x.astype(jnp.bfloat16)


def _fused_kernel(a0_ref, a1_ref, xbf_ref, xi_ref, b1_ref, w1_ref, b2_ref,
                  w2_ref, w3_ref, b3_ref, o_ref, x1f_ref, x1bf_ref):
    s = pl.program_id(0)

    @pl.when(s < N_I)
    def _stage1():
        for h, a_ref in enumerate((a0_ref, a1_ref)):
            agg = jnp.dot(_bf(a_ref[...]), xbf_ref[...],
                          preferred_element_type=jnp.float32)
            pre = agg - 3.0 * xi_ref[pl.ds(s * BM + h * BH, BH), :]
            hh = jnp.dot(_bf(pre), _bf(w1_ref[...]),
                         preferred_element_type=jnp.float32) + b1_ref[...]
            x1 = jnp.maximum(hh, 0.0)
            x1f_ref[pl.ds(s * BM + h * BH, BH), :] = x1
            x1bf_ref[pl.ds(s * BM + h * BH, BH), :] = _bf(x1)

    @pl.when(s >= N_I)
    def _stage2():
        i = s - N_I
        for h, a_ref in enumerate((a0_ref, a1_ref)):
            agg = jnp.dot(_bf(a_ref[...]), x1bf_ref[...],
                          preferred_element_type=jnp.float32)
            pre = agg - 2.0 * x1f_ref[pl.ds(i * BM + h * BH, BH), :]
            hh = jnp.dot(_bf(pre), _bf(w2_ref[...]),
                         preferred_element_type=jnp.float32) + b2_ref[...]
            x2 = jnp.maximum(hh, 0.0)
            o_ref[pl.ds(i * BM + h * BH, BH), :] = jnp.dot(
                _bf(x2), _bf(w3_ref[...]),
                preferred_element_type=jnp.float32) + b3_ref[...]


def kernel(A, X, W1, b1, W2, b2, W3, b3):
    D = X.shape[1]
    H1 = W1.shape[1]
    H2 = W2.shape[1]
    x_bf = X.astype(jnp.bfloat16)

    return pl.pallas_call(
        _fused_kernel,
        grid=(2 * N_I,),
        in_specs=[
            pl.BlockSpec((BH, N), lambda s: (2 * (s % N_I), 0)),
            pl.BlockSpec((BH, N), lambda s: (2 * (s % N_I) + 1, 0)),
            pl.BlockSpec((N, D), lambda s: (0, 0)),
            pl.BlockSpec((N, D), lambda s: (0, 0)),
            pl.BlockSpec((1, H1), lambda s: (0, 0)),
            pl.BlockSpec((D, H1), lambda s: (0, 0)),
            pl.BlockSpec((1, H2), lambda s: (0, 0)),
            pl.BlockSpec((H1, H2), lambda s: (0, 0)),
            pl.BlockSpec((H2, 1), lambda s: (0, 0)),
            pl.BlockSpec((1, 1), lambda s: (0, 0)),
        ],
        out_specs=pl.BlockSpec((N, 1), lambda s: (0, 0)),
        out_shape=jax.ShapeDtypeStruct((N, 1), jnp.float32),
        scratch_shapes=[
            pltpu.VMEM((N, H1), jnp.float32),
            pltpu.VMEM((N, H1), jnp.bfloat16),
        ],
        compiler_params=pltpu.CompilerParams(
            dimension_semantics=("arbitrary",),
        ),
    )(A, A, x_bf, X, b1.reshape(1, -1), W1, b2.reshape(1, -1), W2, W3,
      b3.reshape(1, 1))


# speedup vs baseline: 1.0056x; 1.0056x over previous
import functools

import jax
import jax.numpy as jnp
from jax.experimental import pallas as pl
from jax.experimental.pallas import tpu as pltpu

N = 10000

BM = 200
N_I = N // BM
NBUF = 4


def _bf(x):
    return x.astype(jnp.bfloat16)


def _fused_kernel(a_hbm, xbf_ref, xf_ref, b1_ref, w1_ref, b2_ref, w2_ref,
                  w3_ref, b3_ref, o_ref, abuf, x1f_ref, x1bf_ref, sems):
    s = pl.program_id(0)
    total = 2 * N_I

    def copy_for(t):
        return pltpu.make_async_copy(
            a_hbm.at[pl.ds((t % N_I) * BM, BM), :],
            abuf.at[t % NBUF],
            sems.at[t % NBUF],
        )

    @pl.when(s == 0)
    def _prime():
        for t in range(NBUF - 1):
            copy_for(t).start()

    @pl.when(s + NBUF - 1 < total)
    def _prefetch():
        copy_for(s + NBUF - 1).start()

    copy_for(s).wait()
    a = abuf[s % NBUF]

    @pl.when(s < N_I)
    def _stage1():
        agg = jnp.dot(_bf(a), xbf_ref[...], preferred_element_type=jnp.float32)
        pre = agg - 3.0 * xf_ref[pl.ds(s * BM, BM), :]
        hh = jnp.dot(_bf(pre), _bf(w1_ref[...]),
                     preferred_element_type=jnp.float32) + b1_ref[...]
        x1 = jnp.maximum(hh, 0.0)
        x1f_ref[pl.ds(s * BM, BM), :] = x1
        x1bf_ref[pl.ds(s * BM, BM), :] = _bf(x1)

    @pl.when(s >= N_I)
    def _stage2():
        i = s - N_I
        agg = jnp.dot(_bf(a), x1bf_ref[...], preferred_element_type=jnp.float32)
        pre = agg - 2.0 * x1f_ref[pl.ds(i * BM, BM), :]
        hh = jnp.dot(_bf(pre), _bf(w2_ref[...]),
                     preferred_element_type=jnp.float32) + b2_ref[...]
        x2 = jnp.maximum(hh, 0.0)
        o_ref[pl.ds(i * BM, BM), :] = jnp.dot(
            _bf(x2), _bf(w3_ref[...]),
            preferred_element_type=jnp.float32) + b3_ref[...]


def kernel(A, X, W1, b1, W2, b2, W3, b3):
    D = X.shape[1]
    H1 = W1.shape[1]
    H2 = W2.shape[1]
    x_bf = X.astype(jnp.bfloat16)

    return pl.pallas_call(
        _fused_kernel,
        grid=(2 * N_I,),
        in_specs=[
            pl.BlockSpec(memory_space=pltpu.MemorySpace.HBM),
            pl.BlockSpec((N, D), lambda s: (0, 0)),
            pl.BlockSpec((N, D), lambda s: (0, 0)),
            pl.BlockSpec((1, H1), lambda s: (0, 0)),
            pl.BlockSpec((D, H1), lambda s: (0, 0)),
            pl.BlockSpec((1, H2), lambda s: (0, 0)),
            pl.BlockSpec((H1, H2), lambda s: (0, 0)),
            pl.BlockSpec((H2, 1), lambda s: (0, 0)),
            pl.BlockSpec((1, 1), lambda s: (0, 0)),
        ],
        out_specs=pl.BlockSpec((N, 1), lambda s: (0, 0)),
        out_shape=jax.ShapeDtypeStruct((N, 1), jnp.float32),
        scratch_shapes=[
            pltpu.VMEM((NBUF, BM, N), jnp.float32),
            pltpu.VMEM((N, H1), jnp.float32),
            pltpu.VMEM((N, H1), jnp.bfloat16),
            pltpu.SemaphoreType.DMA((NBUF,)),
        ],
        compiler_params=pltpu.CompilerParams(
            dimension_semantics=("arbitrary",),
        ),
    )(A, x_bf, X, b1.reshape(1, -1), W1, b2.reshape(1, -1), W2, W3,
      b3.reshape(1, 1))


# speedup vs baseline: 1.0170x; 1.0114x over previous
import functools

import jax
import jax.numpy as jnp
from jax.experimental import pallas as pl
from jax.experimental.pallas import tpu as pltpu

N = 10000

BM = 200
N_I = N // BM
NBUF = 4
SPLIT = 5
BS = BM // SPLIT


def _bf(x):
    return x.astype(jnp.bfloat16)


def _fused_kernel(a_hbm, xf_ref, b1_ref, w1_ref, b2_ref, w2_ref,
                  w3_ref, b3_ref, o_ref, abuf, xbf_ref, x1f_ref, x1bf_ref,
                  sems):
    s = pl.program_id(0)
    total = 2 * N_I

    def copies_for(t):
        return [
            pltpu.make_async_copy(
                a_hbm.at[pl.ds((t % N_I) * BM + h * BS, BS), :],
                abuf.at[t % NBUF, pl.ds(h * BS, BS), :],
                sems.at[t % NBUF, h],
            )
            for h in range(SPLIT)
        ]

    @pl.when(s == 0)
    def _prime():
        for t in range(NBUF - 1):
            for c in copies_for(t):
                c.start()
        xbf_ref[...] = _bf(xf_ref[...])

    @pl.when(s + NBUF - 1 < total)
    def _prefetch():
        for c in copies_for(s + NBUF - 1):
            c.start()

    for c in copies_for(s):
        c.wait()
    a = abuf[s % NBUF]

    @pl.when(s < N_I)
    def _stage1():
        agg = jnp.dot(_bf(a), xbf_ref[...], preferred_element_type=jnp.float32)
        pre = agg - 3.0 * xf_ref[pl.ds(s * BM, BM), :]
        hh = jnp.dot(_bf(pre), _bf(w1_ref[...]),
                     preferred_element_type=jnp.float32) + b1_ref[...]
        x1 = jnp.maximum(hh, 0.0)
        x1f_ref[pl.ds(s * BM, BM), :] = x1
        x1bf_ref[pl.ds(s * BM, BM), :] = _bf(x1)

    @pl.when(s >= N_I)
    def _stage2():
        i = s - N_I
        agg = jnp.dot(_bf(a), x1bf_ref[...], preferred_element_type=jnp.float32)
        pre = agg - 2.0 * x1f_ref[pl.ds(i * BM, BM), :]
        hh = jnp.dot(_bf(pre), _bf(w2_ref[...]),
                     preferred_element_type=jnp.float32) + b2_ref[...]
        x2 = jnp.maximum(hh, 0.0)
        o_ref[pl.ds(i * BM, BM), :] = jnp.dot(
            _bf(x2), _bf(w3_ref[...]),
            preferred_element_type=jnp.float32) + b3_ref[...]


def kernel(A, X, W1, b1, W2, b2, W3, b3):
    D = X.shape[1]
    H1 = W1.shape[1]
    H2 = W2.shape[1]

    return pl.pallas_call(
        _fused_kernel,
        grid=(2 * N_I,),
        in_specs=[
            pl.BlockSpec(memory_space=pltpu.MemorySpace.HBM),
            pl.BlockSpec((N, D), lambda s: (0, 0)),
            pl.BlockSpec((1, H1), lambda s: (0, 0)),
            pl.BlockSpec((D, H1), lambda s: (0, 0)),
            pl.BlockSpec((1, H2), lambda s: (0, 0)),
            pl.BlockSpec((H1, H2), lambda s: (0, 0)),
            pl.BlockSpec((H2, 1), lambda s: (0, 0)),
            pl.BlockSpec((1, 1), lambda s: (0, 0)),
        ],
        out_specs=pl.BlockSpec((N, 1), lambda s: (0, 0)),
        out_shape=jax.ShapeDtypeStruct((N, 1), jnp.float32),
        scratch_shapes=[
            pltpu.VMEM((NBUF, BM, N), jnp.float32),
            pltpu.VMEM((N, D), jnp.bfloat16),
            pltpu.VMEM((N, H1), jnp.float32),
            pltpu.VMEM((N, H1), jnp.bfloat16),
            pltpu.SemaphoreType.DMA((NBUF, SPLIT)),
        ],
        compiler_params=pltpu.CompilerParams(
            dimension_semantics=("arbitrary",),
        ),
    )(A, X, b1.reshape(1, -1), W1, b2.reshape(1, -1), W2, W3,
      b3.reshape(1, 1))


# speedup vs baseline: 1.0203x; 1.0032x over previous
import functools

import jax
import jax.numpy as jnp
from jax.experimental import pallas as pl
from jax.experimental.pallas import tpu as pltpu

N = 10000

BM = 200
N_I = N // BM
NBUF = 4
SPLIT = 5
BS = BM // SPLIT


def _bf(x):
    return x.astype(jnp.bfloat16)


def _fused_kernel(a_hbm, x_hbm, b1_ref, w1_ref, b2_ref, w2_ref,
                  w3_ref, b3_ref, o_ref, abuf, xf_ref, xbf_ref, x1f_ref,
                  x1bf_ref, sems, xsem):
    s = pl.program_id(0)
    total = 2 * N_I

    def copies_for(t):
        return [
            pltpu.make_async_copy(
                a_hbm.at[pl.ds((t % N_I) * BM + h * BS, BS), :],
                abuf.at[t % NBUF, pl.ds(h * BS, BS), :],
                sems.at[t % NBUF, h],
            )
            for h in range(SPLIT)
        ]

    @pl.when(s == 0)
    def _prime():
        for t in range(NBUF - 1):
            for c in copies_for(t):
                c.start()
        xcopy = pltpu.make_async_copy(x_hbm, xf_ref, xsem)
        xcopy.start()
        xcopy.wait()
        xbf_ref[...] = _bf(xf_ref[...])

    @pl.when(s + NBUF - 1 < total)
    def _prefetch():
        for c in copies_for(s + NBUF - 1):
            c.start()

    for c in copies_for(s):
        c.wait()
    a = abuf[s % NBUF]

    @pl.when(s < N_I)
    def _stage1():
        agg = jnp.dot(_bf(a), xbf_ref[...], preferred_element_type=jnp.float32)
        pre = agg - 3.0 * xf_ref[pl.ds(s * BM, BM), :]
        hh = jnp.dot(_bf(pre), _bf(w1_ref[...]),
                     preferred_element_type=jnp.float32) + b1_ref[...]
        x1 = jnp.maximum(hh, 0.0)
        x1f_ref[pl.ds(s * BM, BM), :] = x1
        x1bf_ref[pl.ds(s * BM, BM), :] = _bf(x1)

    @pl.when(s >= N_I)
    def _stage2():
        i = s - N_I
        agg = jnp.dot(_bf(a), x1bf_ref[...], preferred_element_type=jnp.float32)
        pre = agg - 2.0 * x1f_ref[pl.ds(i * BM, BM), :]
        hh = jnp.dot(_bf(pre), _bf(w2_ref[...]),
                     preferred_element_type=jnp.float32) + b2_ref[...]
        x2 = jnp.maximum(hh, 0.0)
        o_ref[pl.ds(i * BM, BM), :] = jnp.dot(
            _bf(x2), _bf(w3_ref[...]),
            preferred_element_type=jnp.float32) + b3_ref[...]


def kernel(A, X, W1, b1, W2, b2, W3, b3):
    D = X.shape[1]
    H1 = W1.shape[1]
    H2 = W2.shape[1]

    return pl.pallas_call(
        _fused_kernel,
        grid=(2 * N_I,),
        in_specs=[
            pl.BlockSpec(memory_space=pltpu.MemorySpace.HBM),
            pl.BlockSpec(memory_space=pltpu.MemorySpace.HBM),
            pl.BlockSpec((1, H1), lambda s: (0, 0)),
            pl.BlockSpec((D, H1), lambda s: (0, 0)),
            pl.BlockSpec((1, H2), lambda s: (0, 0)),
            pl.BlockSpec((H1, H2), lambda s: (0, 0)),
            pl.BlockSpec((H2, 1), lambda s: (0, 0)),
            pl.BlockSpec((1, 1), lambda s: (0, 0)),
        ],
        out_specs=pl.BlockSpec((N, 1), lambda s: (0, 0)),
        out_shape=jax.ShapeDtypeStruct((N, 1), jnp.float32),
        scratch_shapes=[
            pltpu.VMEM((NBUF, BM, N), jnp.float32),
            pltpu.VMEM((N, D), jnp.float32),
            pltpu.VMEM((N, D), jnp.bfloat16),
            pltpu.VMEM((N, H1), jnp.float32),
            pltpu.VMEM((N, H1), jnp.bfloat16),
            pltpu.SemaphoreType.DMA((NBUF, SPLIT)),
            pltpu.SemaphoreType.DMA,
        ],
        compiler_params=pltpu.CompilerParams(
            dimension_semantics=("arbitrary",),
            vmem_limit_bytes=66 * 1024 * 1024,
        ),
    )(A, X, b1.reshape(1, -1), W1, b2.reshape(1, -1), W2, W3,
      b3.reshape(1, 1))
